# baseline (speedup 1.0000x reference)
import functools

import jax
import jax.numpy as jnp
from jax.experimental import pallas as pl
from jax.experimental.pallas import tpu as pltpu


def _scn_block(h_ref, x_ref, w_ref, g_ref, b_ref, o_ref, u_s, inv_s, hc_s,
               *, R, K, S):
    p = pl.program_id(0)
    i = pl.program_id(1)

    @pl.when(p == 0)
    def _rowsum_phase():
        strip = h_ref[...]
        s = jnp.sum(jnp.abs(strip), axis=1, keepdims=True)
        inv = jnp.where(s > 0, jax.lax.rsqrt(s), 0.0)
        inv_s[pl.ds(i * R, R), :] = inv
        z = jnp.dot(x_ref[...], w_ref[...],
                    preferred_element_type=jnp.float32)
        u_s[pl.ds(i * R, R), :] = (inv * z).astype(jnp.bfloat16)

        if K > 0:
            @pl.when(i < K)
            def _cache():
                ic = jnp.minimum(i, K - 1)
                hc_s[ic] = strip.astype(jnp.bfloat16)

    def _epilogue(acc):
        inv = inv_s[pl.ds(i * R, R), :]
        h = jax.nn.relu(acc * inv) + x_ref[...]
        mu = jnp.mean(h, axis=1, keepdims=True)
        var = jnp.mean((h - mu) ** 2, axis=1, keepdims=True)
        o_ref[...] = ((h - mu) * jax.lax.rsqrt(var + 1e-5)
                      * g_ref[...] + b_ref[...])

    for k in range(K):
        @pl.when((p == 1) & (i == k))
        def _matmul_cached(k=k):
            strip = hc_s[k]
            _epilogue(jax.lax.dot_general(
                strip, u_s[...], (((1,), (0,)), ((), ())),
                preferred_element_type=jnp.float32))

    @pl.when((p == 1) & (i >= K))
    def _matmul_streamed():
        strip = h_ref[...].astype(jnp.bfloat16)
        _epilogue(jax.lax.dot_general(
            strip, u_s[...], (((1,), (0,)), ((), ())),
            preferred_element_type=jnp.float32))


def _scn_layer(h, x, w, g, b, R, K):
    n, d = x.shape
    S = n // R
    grid = (2, S)

    def h_map(p, i):
        return (jnp.where(p == 0, i, jnp.where(i >= K, i, S - 1)), 0)

    return pl.pallas_call(
        functools.partial(_scn_block, R=R, K=K, S=S),
        grid=grid,
        in_specs=[
            pl.BlockSpec((R, n), h_map),
            pl.BlockSpec((R, d), lambda p, i: (i, 0)),
            pl.BlockSpec((d, d), lambda p, i: (0, 0)),
            pl.BlockSpec((1, d), lambda p, i: (0, 0)),
            pl.BlockSpec((1, d), lambda p, i: (0, 0)),
        ],
        out_specs=pl.BlockSpec((R, d), lambda p, i: (i * p, 0)),
        out_shape=jax.ShapeDtypeStruct((n, d), jnp.float32),
        scratch_shapes=[
            pltpu.VMEM((n, d), jnp.bfloat16),
            pltpu.VMEM((n, 1), jnp.float32),
            pltpu.VMEM((K, R, n) if K > 0 else (1, 8, 128), jnp.bfloat16),
        ],
    )(h, x, w, g.reshape(1, d), b.reshape(1, d))


def kernel(x_0, x_1, x_2, hodge_laplacian_0, hodge_laplacian_1,
           hodge_laplacian_2, y, batch_0, W0, W1, W2,
           ln0_g, ln0_b, ln1_g, ln1_b, ln2_g, ln2_b):
    out0 = _scn_layer(hodge_laplacian_0, x_0, W0, ln0_g, ln0_b, R=512, K=7)
    out1 = _scn_layer(hodge_laplacian_1, x_1, W1, ln1_g, ln1_b, R=512, K=0)
    out2 = _scn_layer(hodge_laplacian_2, x_2, W2, ln2_g, ln2_b, R=512, K=7)
    return (out0, out1, out2)


# speedup vs baseline: 1.0051x; 1.0051x over previous
import functools

import jax
import jax.numpy as jnp
from jax.experimental import pallas as pl
from jax.experimental.pallas import tpu as pltpu


def _scn_block(h_ref, x_ref, w_ref, g_ref, b_ref, o_ref, u_s, inv_s, hc_s,
               *, R, K, S):
    p = pl.program_id(0)
    i = pl.program_id(1)

    @pl.when(p == 0)
    def _rowsum_phase():
        strip = h_ref[...]
        s = jnp.sum(jnp.abs(strip), axis=1, keepdims=True)
        inv = jnp.where(s > 0, jax.lax.rsqrt(s), 0.0)
        inv_s[pl.ds(i * R, R), :] = inv
        z = jnp.dot(x_ref[...], w_ref[...],
                    preferred_element_type=jnp.float32)
        u_s[pl.ds(i * R, R), :] = (inv * z).astype(jnp.bfloat16)

        if K > 0:
            @pl.when(i < K)
            def _cache():
                ic = jnp.minimum(i, K - 1)
                hc_s[ic] = strip.astype(jnp.bfloat16)

    def _epilogue(acc):
        inv = inv_s[pl.ds(i * R, R), :]
        h = jax.nn.relu(acc * inv) + x_ref[...]
        mu = jnp.mean(h, axis=1, keepdims=True)
        var = jnp.mean((h - mu) ** 2, axis=1, keepdims=True)
        o_ref[...] = ((h - mu) * jax.lax.rsqrt(var + 1e-5)
                      * g_ref[...] + b_ref[...])

    if K > 0:
        @pl.when((p == 1) & (i < K))
        def _matmul_cached():
            ic = jnp.minimum(i, K - 1)
            strip = hc_s[ic]
            _epilogue(jax.lax.dot_general(
                strip, u_s[...], (((1,), (0,)), ((), ())),
                preferred_element_type=jnp.float32))

    @pl.when((p == 1) & (i >= K))
    def _matmul_streamed():
        strip = h_ref[...].astype(jnp.bfloat16)
        _epilogue(jax.lax.dot_general(
            strip, u_s[...], (((1,), (0,)), ((), ())),
            preferred_element_type=jnp.float32))


def _scn_layer(h, x, w, g, b, R, K):
    n, d = x.shape
    S = n // R
    grid = (2, S)

    def h_map(p, i):
        return (jnp.where(p == 0, i, jnp.where(i >= K, i, S - 1)), 0)

    return pl.pallas_call(
        functools.partial(_scn_block, R=R, K=K, S=S),
        grid=grid,
        in_specs=[
            pl.BlockSpec((R, n), h_map),
            pl.BlockSpec((R, d), lambda p, i: (i, 0)),
            pl.BlockSpec((d, d), lambda p, i: (0, 0)),
            pl.BlockSpec((1, d), lambda p, i: (0, 0)),
            pl.BlockSpec((1, d), lambda p, i: (0, 0)),
        ],
        out_specs=pl.BlockSpec((R, d), lambda p, i: (i * p, 0)),
        out_shape=jax.ShapeDtypeStruct((n, d), jnp.float32),
        scratch_shapes=[
            pltpu.VMEM((n, d), jnp.bfloat16),
            pltpu.VMEM((n, 1), jnp.float32),
            pltpu.VMEM((K, R, n) if K > 0 else (1, 8, 128), jnp.bfloat16),
        ],
    )(h, x, w, g.reshape(1, d), b.reshape(1, d))


def kernel(x_0, x_1, x_2, hodge_laplacian_0, hodge_laplacian_1,
           hodge_laplacian_2, y, batch_0, W0, W1, W2,
           ln0_g, ln0_b, ln1_g, ln1_b, ln2_g, ln2_b):
    out0 = _scn_layer(hodge_laplacian_0, x_0, W0, ln0_g, ln0_b, R=512, K=7)
    out1 = _scn_layer(hodge_laplacian_1, x_1, W1, ln1_g, ln1_b, R=512, K=0)
    out2 = _scn_layer(hodge_laplacian_2, x_2, W2, ln2_g, ln2_b, R=512, K=7)
    return (out0, out1, out2)


# speedup vs baseline: 1.0151x; 1.0099x over previous
import functools

import jax
import jax.numpy as jnp
from jax.experimental import pallas as pl
from jax.experimental.pallas import tpu as pltpu


def _scn_block(h_ref, x_ref, w_ref, g_ref, b_ref, o_ref, u_s, inv_s, hc_s,
               *, R, K, S, rev):
    p = pl.program_id(0)
    gi = pl.program_id(1)
    i = jnp.where((p == 1) & rev, S - 1 - gi, gi) if rev else gi

    @pl.when(p == 0)
    def _rowsum_phase():
        strip = h_ref[...]
        s = jnp.sum(jnp.abs(strip), axis=1, keepdims=True)
        inv = jnp.where(s > 0, jax.lax.rsqrt(s), 0.0)
        inv_s[pl.ds(i * R, R), :] = inv
        z = jnp.dot(x_ref[...], w_ref[...],
                    preferred_element_type=jnp.float32)
        u_s[pl.ds(i * R, R), :] = (inv * z).astype(jnp.bfloat16)

        if K > 0:
            @pl.when(i < K)
            def _cache():
                ic = jnp.minimum(i, K - 1)
                hc_s[ic] = strip.astype(jnp.bfloat16)

    def _epilogue(acc):
        inv = inv_s[pl.ds(i * R, R), :]
        h = jax.nn.relu(acc * inv) + x_ref[...]
        mu = jnp.mean(h, axis=1, keepdims=True)
        var = jnp.mean((h - mu) ** 2, axis=1, keepdims=True)
        o_ref[...] = ((h - mu) * jax.lax.rsqrt(var + 1e-5)
                      * g_ref[...] + b_ref[...])

    if K > 0:
        @pl.when((p == 1) & (i < K))
        def _matmul_cached():
            ic = jnp.minimum(i, K - 1)
            strip = hc_s[ic]
            _epilogue(jax.lax.dot_general(
                strip, u_s[...], (((1,), (0,)), ((), ())),
                preferred_element_type=jnp.float32))

    @pl.when((p == 1) & (i >= K))
    def _matmul_streamed():
        strip = h_ref[...].astype(jnp.bfloat16)
        _epilogue(jax.lax.dot_general(
            strip, u_s[...], (((1,), (0,)), ((), ())),
            preferred_element_type=jnp.float32))


def _scn_layer(h, x, w, g, b, R, K):
    n, d = x.shape
    S = n // R
    grid = (2, S)

    rev = (K == 0)

    def h_map(p, i):
        j = jnp.where((p == 1) & rev, S - 1 - i, i)
        return (jnp.where(p == 0, j, jnp.where(j >= K, j, S - 1)), 0)

    def xo_map(p, i):
        return (jnp.where((p == 1) & rev, S - 1 - i, i), 0)

    return pl.pallas_call(
        functools.partial(_scn_block, R=R, K=K, S=S, rev=rev),
        grid=grid,
        in_specs=[
            pl.BlockSpec((R, n), h_map),
            pl.BlockSpec((R, d), xo_map),
            pl.BlockSpec((d, d), lambda p, i: (0, 0)),
            pl.BlockSpec((1, d), lambda p, i: (0, 0)),
            pl.BlockSpec((1, d), lambda p, i: (0, 0)),
        ],
        out_specs=pl.BlockSpec((R, d), lambda p, i: (xo_map(p, i)[0] * p, 0)),
        out_shape=jax.ShapeDtypeStruct((n, d), jnp.float32),
        scratch_shapes=[
            pltpu.VMEM((n, d), jnp.bfloat16),
            pltpu.VMEM((n, 1), jnp.float32),
            pltpu.VMEM((K, R, n) if K > 0 else (1, 8, 128), jnp.bfloat16),
        ],
    )(h, x, w, g.reshape(1, d), b.reshape(1, d))


def kernel(x_0, x_1, x_2, hodge_laplacian_0, hodge_laplacian_1,
           hodge_laplacian_2, y, batch_0, W0, W1, W2,
           ln0_g, ln0_b, ln1_g, ln1_b, ln2_g, ln2_b):
    out0 = _scn_layer(hodge_laplacian_0, x_0, W0, ln0_g, ln0_b, R=512, K=7)
    out1 = _scn_layer(hodge_laplacian_1, x_1, W1, ln1_g, ln1_b, R=512, K=0)
    out2 = _scn_layer(hodge_laplacian_2, x_2, W2, ln2_g, ln2_b, R=512, K=7)
    return (out0, out1, out2)
